# Initial kernel scaffold; baseline (speedup 1.0000x reference)
import jax
import jax.numpy as jnp
from jax.experimental import pallas as pl


def kernel(x, gamma, beta, alpha):
    raise NotImplementedError("write your pallas kernel here")


# speedup vs baseline: 1.1038x; 1.1038x over previous
import functools

import jax
import jax.numpy as jnp
from jax.experimental import pallas as pl
from jax.experimental.pallas import tpu as pltpu

_CORES = 2


def _stats_kernel(x_ref, sum_ref, sq_ref):
    i = pl.program_id(1)
    x = x_ref[...]
    s = jnp.sum(x, axis=0, keepdims=True)
    sq = jnp.sum(x * x, axis=0, keepdims=True)

    @pl.when(i == 0)
    def _():
        sum_ref[0] = s
        sq_ref[0] = sq

    @pl.when(i > 0)
    def _():
        sum_ref[0] = sum_ref[0] + s
        sq_ref[0] = sq_ref[0] + sq


def _apply_kernel(x_ref, sum_ref, sq_ref, g_ref, b_ref, a_ref, o_ref, *,
                  inv_n, eps):
    s = jnp.sum(sum_ref[...], axis=0)
    sq = jnp.sum(sq_ref[...], axis=0)
    mean = s * inv_n
    var = jnp.maximum(sq * inv_n - mean * mean, 0.0)
    scale = g_ref[...] * jax.lax.rsqrt(var + eps)
    shift = b_ref[...] - mean * scale
    a = a_ref[...]
    x = x_ref[...]
    p = jax.nn.sigmoid(x * scale + shift)
    o_ref[...] = x * (a + p * (1.0 - a))


@functools.partial(jax.jit, static_argnames=("eps",))
def _dice(x, gamma, beta, alpha, *, eps):
    n, e = x.shape
    out_dtype = x.dtype

    tn = 2048
    ntot = _CORES * tn
    pad = (-n) % ntot
    xp = jnp.pad(x, ((0, pad), (0, 0))) if pad else x
    n_rows = n + pad
    nt = n_rows // tn
    per_core = nt // _CORES

    g2 = gamma.reshape(1, e).astype(jnp.float32)
    b2 = beta.reshape(1, e).astype(jnp.float32)
    a2 = alpha.reshape(1, e).astype(jnp.float32)

    s_sum, s_sq = pl.pallas_call(
        _stats_kernel,
        out_shape=(jax.ShapeDtypeStruct((_CORES, 1, e), jnp.float32),
                   jax.ShapeDtypeStruct((_CORES, 1, e), jnp.float32)),
        grid=(_CORES, per_core),
        in_specs=[pl.BlockSpec((tn, e), lambda c, i: (c * per_core + i, 0))],
        out_specs=(pl.BlockSpec((1, 1, e), lambda c, i: (c, 0, 0)),
                   pl.BlockSpec((1, 1, e), lambda c, i: (c, 0, 0))),
        compiler_params=pltpu.CompilerParams(
            dimension_semantics=("parallel", "arbitrary")),
    )(xp)

    apply_kernel = functools.partial(_apply_kernel, inv_n=1.0 / n, eps=eps)
    out_p = pl.pallas_call(
        apply_kernel,
        out_shape=jax.ShapeDtypeStruct((n_rows, e), out_dtype),
        grid=(nt,),
        in_specs=[pl.BlockSpec((tn, e), lambda i: (i, 0)),
                  pl.BlockSpec((_CORES, 1, e), lambda i: (0, 0, 0)),
                  pl.BlockSpec((_CORES, 1, e), lambda i: (0, 0, 0)),
                  pl.BlockSpec((1, e), lambda i: (0, 0)),
                  pl.BlockSpec((1, e), lambda i: (0, 0)),
                  pl.BlockSpec((1, e), lambda i: (0, 0))],
        out_specs=pl.BlockSpec((tn, e), lambda i: (i, 0)),
        compiler_params=pltpu.CompilerParams(
            dimension_semantics=("parallel",)),
    )(xp, s_sum, s_sq, g2, b2, a2)

    return out_p[:n] if pad else out_p


def kernel(x, gamma, beta, alpha):
    return _dice(x, gamma, beta, alpha, eps=1e-8)


# speedup vs baseline: 1.1844x; 1.0730x over previous
import functools

import jax
import jax.numpy as jnp
from jax.experimental import pallas as pl
from jax.experimental.pallas import tpu as pltpu

_CORES = 2


def _stats_kernel(x_ref, sum_ref, sq_ref):
    i = pl.program_id(1)
    x = x_ref[...]
    s = jnp.sum(x, axis=0, keepdims=True)
    sq = jnp.sum(x * x, axis=0, keepdims=True)

    @pl.when(i == 0)
    def _():
        sum_ref[0] = s
        sq_ref[0] = sq

    @pl.when(i > 0)
    def _():
        sum_ref[0] = sum_ref[0] + s
        sq_ref[0] = sq_ref[0] + sq


def _apply_kernel(x_ref, sum_ref, sq_ref, g_ref, b_ref, a_ref, o_ref, *,
                  inv_n, eps):
    s = jnp.sum(sum_ref[...], axis=0)
    sq = jnp.sum(sq_ref[...], axis=0)
    mean = s * inv_n
    var = jnp.maximum(sq * inv_n - mean * mean, 0.0)
    scale = g_ref[...] * jax.lax.rsqrt(var + eps)
    shift = b_ref[...] - mean * scale
    a = a_ref[...]
    x = x_ref[...]
    p = jax.nn.sigmoid(x * scale + shift)
    o_ref[...] = x * (a + p * (1.0 - a))


@functools.partial(jax.jit, static_argnames=("eps",))
def _dice(x, gamma, beta, alpha, *, eps):
    n, e = x.shape
    out_dtype = x.dtype

    tn = 4096
    ntot = _CORES * tn
    pad = (-n) % ntot
    xp = jnp.pad(x, ((0, pad), (0, 0))) if pad else x
    n_rows = n + pad
    nt = n_rows // tn
    per_core = nt // _CORES

    g2 = gamma.reshape(1, e).astype(jnp.float32)
    b2 = beta.reshape(1, e).astype(jnp.float32)
    a2 = alpha.reshape(1, e).astype(jnp.float32)

    s_sum, s_sq = pl.pallas_call(
        _stats_kernel,
        out_shape=(jax.ShapeDtypeStruct((_CORES, 1, e), jnp.float32),
                   jax.ShapeDtypeStruct((_CORES, 1, e), jnp.float32)),
        grid=(_CORES, per_core),
        in_specs=[pl.BlockSpec((tn, e), lambda c, i: (c * per_core + i, 0))],
        out_specs=(pl.BlockSpec((1, 1, e), lambda c, i: (c, 0, 0)),
                   pl.BlockSpec((1, 1, e), lambda c, i: (c, 0, 0))),
        compiler_params=pltpu.CompilerParams(
            dimension_semantics=("parallel", "arbitrary")),
    )(xp)

    apply_kernel = functools.partial(_apply_kernel, inv_n=1.0 / n, eps=eps)
    out_p = pl.pallas_call(
        apply_kernel,
        out_shape=jax.ShapeDtypeStruct((n_rows, e), out_dtype),
        grid=(nt,),
        in_specs=[pl.BlockSpec((tn, e), lambda i: (i, 0)),
                  pl.BlockSpec((_CORES, 1, e), lambda i: (0, 0, 0)),
                  pl.BlockSpec((_CORES, 1, e), lambda i: (0, 0, 0)),
                  pl.BlockSpec((1, e), lambda i: (0, 0)),
                  pl.BlockSpec((1, e), lambda i: (0, 0)),
                  pl.BlockSpec((1, e), lambda i: (0, 0))],
        out_specs=pl.BlockSpec((tn, e), lambda i: (i, 0)),
        compiler_params=pltpu.CompilerParams(
            dimension_semantics=("parallel",)),
    )(xp, s_sum, s_sq, g2, b2, a2)

    return out_p[:n] if pad else out_p


def kernel(x, gamma, beta, alpha):
    return _dice(x, gamma, beta, alpha, eps=1e-8)
